# baseline (speedup 1.0000x reference)
import functools

import jax
import jax.numpy as jnp
from jax import lax
from jax.experimental import pallas as pl
from jax.experimental.pallas import tpu as pltpu
from jax.experimental.pallas import tpu_sc as plsc

V = 1000
VP = 1024
N = 51200
NC, NS = 2, 16
NW = NC * NS
BPW = N // NW
CHUNK = 80
NCHUNK = BPW // CHUNK

BLK = 256
G = N // BLK


def _sc_gather(embedding, idx_flat):
    mesh = plsc.VectorSubcoreMesh(core_axis_name="c", subcore_axis_name="s")

    @functools.partial(
        pl.kernel,
        out_type=jax.ShapeDtypeStruct((N, VP), jnp.float32),
        mesh=mesh,
        scratch_types=[
            pltpu.VMEM((BPW,), jnp.int32),
            pltpu.VMEM((CHUNK, VP), jnp.float32),
            pltpu.SemaphoreType.DMA,
        ],
    )
    def k(table_hbm, idx_hbm, out_hbm, idx_v, rows_v, sem):
        wid = lax.axis_index("s") * NC + lax.axis_index("c")
        base = wid * BPW
        pltpu.sync_copy(idx_hbm.at[pl.ds(base, BPW)], idx_v)

        @pl.loop(0, NCHUNK)
        def _(c):
            off = c * CHUNK
            pltpu.async_copy(
                table_hbm.at[idx_v.at[pl.ds(off, CHUNK)]], rows_v, sem
            ).wait()
            pltpu.sync_copy(rows_v, out_hbm.at[pl.ds(base + off, CHUNK)])

    return k(embedding, idx_flat)


def _tc_ce_sum(padded, targets_col):
    def body(t_ref, x_ref, o_ref, loss_ref):
        i = pl.program_id(0)
        rows = x_ref[...]
        o_ref[...] = rows[:, :V]
        m = jnp.max(rows, axis=1, keepdims=True)
        s = jnp.sum(jnp.exp(rows - m), axis=1, keepdims=True)
        lse = m + jnp.log(s)
        lane = lax.broadcasted_iota(jnp.int32, (BLK, VP), 1)
        val = jnp.max(
            jnp.where(lane == t_ref[...], rows, jnp.float32(-1e30)),
            axis=1, keepdims=True,
        )
        part = jnp.sum(lse - val)

        @pl.when(i == 0)
        def _():
            loss_ref[0, 0] = 0.0

        loss_ref[0, 0] += part

    return pl.pallas_call(
        body,
        grid=(G,),
        in_specs=[
            pl.BlockSpec((BLK, 1), lambda i: (i, 0)),
            pl.BlockSpec((BLK, VP), lambda i: (i, 0)),
        ],
        out_specs=[
            pl.BlockSpec((BLK, V), lambda i: (i, 0)),
            pl.BlockSpec(
                block_shape=(1, 1), index_map=lambda i: (0, 0),
                memory_space=pltpu.SMEM,
            ),
        ],
        out_shape=[
            jax.ShapeDtypeStruct((N, V), jnp.float32),
            jax.ShapeDtypeStruct((1, 1), jnp.float32),
        ],
    )(targets_col, padded)


def kernel(idx, targets, embedding):
    idx_flat = idx.reshape(-1)
    emb_p = jnp.pad(embedding, ((0, 0), (0, VP - V)),
                    constant_values=jnp.float32(-1e30))
    padded = _sc_gather(emb_p, idx_flat)
    logits, loss_sum = _tc_ce_sum(padded, targets.reshape(-1, 1))
    return logits, loss_sum[0, 0] / jnp.float32(N)


# speedup vs baseline: 1.0864x; 1.0864x over previous
import functools

import jax
import jax.numpy as jnp
from jax import lax
from jax.experimental import pallas as pl
from jax.experimental.pallas import tpu as pltpu
from jax.experimental.pallas import tpu_sc as plsc

V = 1000
VP = 1024
N = 51200
NC, NS = 2, 16
NW = NC * NS
BPW = N // NW
CHUNK = 80
NCHUNK = BPW // CHUNK

BLK = 512
G = N // BLK


def _sc_gather(embedding, idx_flat):
    mesh = plsc.VectorSubcoreMesh(core_axis_name="c", subcore_axis_name="s")

    @functools.partial(
        pl.kernel,
        out_type=jax.ShapeDtypeStruct((N, VP), jnp.float32),
        mesh=mesh,
        scratch_types=[
            pltpu.VMEM((BPW,), jnp.int32),
            pltpu.VMEM((CHUNK, VP), jnp.float32),
            pltpu.SemaphoreType.DMA,
        ],
    )
    def k(table_hbm, idx_hbm, out_hbm, idx_v, rows_v, sem):
        wid = lax.axis_index("s") * NC + lax.axis_index("c")
        base = wid * BPW
        pltpu.sync_copy(idx_hbm.at[pl.ds(base, BPW)], idx_v)

        @pl.loop(0, NCHUNK)
        def _(c):
            off = c * CHUNK
            pltpu.async_copy(
                table_hbm.at[idx_v.at[pl.ds(off, CHUNK)]], rows_v, sem
            ).wait()
            pltpu.sync_copy(rows_v, out_hbm.at[pl.ds(base + off, CHUNK)])

    return k(embedding, idx_flat)


def _tc_ce_sum(padded, targets_col):
    def body(t_ref, x_ref, o_ref, loss_ref):
        i = pl.program_id(0)
        rows = x_ref[...]
        o_ref[...] = rows[:, :V]
        m = jnp.max(rows, axis=1, keepdims=True)
        s = jnp.sum(jnp.exp(rows - m), axis=1, keepdims=True)
        lse = m + jnp.log(s)
        lane = lax.broadcasted_iota(jnp.int32, (BLK, VP), 1)
        val = jnp.max(
            jnp.where(lane == t_ref[...], rows, jnp.float32(-1e30)),
            axis=1, keepdims=True,
        )
        part = jnp.sum(lse - val)

        @pl.when((i == 0) | (i == G // 2))
        def _():
            loss_ref[0, 0, 0] = 0.0

        loss_ref[0, 0, 0] += part

    return pl.pallas_call(
        body,
        grid=(G,),
        compiler_params=pltpu.CompilerParams(
            dimension_semantics=("parallel",)),
        in_specs=[
            pl.BlockSpec((BLK, 1), lambda i: (i, 0)),
            pl.BlockSpec((BLK, VP), lambda i: (i, 0)),
        ],
        out_specs=[
            pl.BlockSpec((BLK, V), lambda i: (i, 0)),
            pl.BlockSpec(
                block_shape=(1, 1, 1),
                index_map=lambda i: (i // (G // 2), 0, 0),
                memory_space=pltpu.SMEM,
            ),
        ],
        out_shape=[
            jax.ShapeDtypeStruct((N, V), jnp.float32),
            jax.ShapeDtypeStruct((2, 1, 1), jnp.float32),
        ],
    )(targets_col, padded)


def kernel(idx, targets, embedding):
    idx_flat = idx.reshape(-1)
    emb_p = jnp.pad(embedding, ((0, 0), (0, VP - V)),
                    constant_values=jnp.float32(-1e30))
    padded = _sc_gather(emb_p, idx_flat)
    logits, loss_sum = _tc_ce_sum(padded, targets.reshape(-1, 1))
    return logits, (loss_sum[0, 0, 0] + loss_sum[1, 0, 0]) / jnp.float32(N)


# speedup vs baseline: 1.6143x; 1.4860x over previous
import functools

import jax
import jax.numpy as jnp
from jax import lax
from jax.experimental import pallas as pl
from jax.experimental.pallas import tpu as pltpu
from jax.experimental.pallas import tpu_sc as plsc

V = 1000
VP = 1024
N = 51200
NC, NS = 2, 16
NW = NC * NS
BPW = N // NW
CHUNK = 80
NCHUNK = BPW // CHUNK
L = 16


def _tc_lse_table(embedding):
    def body(x_ref, o_ref):
        rows = x_ref[...]
        m = jnp.max(rows, axis=1, keepdims=True)
        s = jnp.sum(jnp.exp(rows - m), axis=1, keepdims=True)
        o_ref[...] = m + jnp.log(s)

    return pl.pallas_call(
        body,
        in_specs=[pl.BlockSpec((V, V), lambda: (0, 0))],
        out_specs=pl.BlockSpec((V, 1), lambda: (0, 0)),
        out_shape=jax.ShapeDtypeStruct((V, 1), jnp.float32),
    )(embedding)


def _sc_gather_and_nll(table_p, emb_flat, lse_flat, idx_flat, tgt_flat):
    mesh = plsc.VectorSubcoreMesh(core_axis_name="c", subcore_axis_name="s")

    @functools.partial(
        pl.kernel,
        out_type=[
            jax.ShapeDtypeStruct((N, VP), jnp.float32),
            jax.ShapeDtypeStruct((NW, L), jnp.float32),
        ],
        mesh=mesh,
        scratch_types=[
            pltpu.VMEM((BPW,), jnp.int32),
            pltpu.VMEM((BPW,), jnp.int32),
            pltpu.VMEM((BPW,), jnp.float32),
            pltpu.VMEM((BPW,), jnp.float32),
            pltpu.VMEM((CHUNK, VP), jnp.float32),
            pltpu.VMEM((L,), jnp.float32),
            pltpu.SemaphoreType.DMA,
            pltpu.SemaphoreType.DMA,
        ],
    )
    def k(table_hbm, flat_hbm, lse_hbm, idx_hbm, tgt_hbm, out_hbm, part_hbm,
          idx_v, pos_v, lsev_v, val_v, rows_v, acc_v, sem, sem2):
        wid = lax.axis_index("s") * NC + lax.axis_index("c")
        base = wid * BPW
        pltpu.sync_copy(idx_hbm.at[pl.ds(base, BPW)], idx_v)
        pltpu.sync_copy(tgt_hbm.at[pl.ds(base, BPW)], pos_v)

        @pl.loop(0, BPW, step=L)
        def _(i):
            sl = pl.ds(i, L)
            pos_v[sl] = idx_v[sl] * V + pos_v[sl]

        lse_dma = pltpu.async_copy(lse_hbm.at[idx_v], lsev_v, sem2)
        val_dma = pltpu.async_copy(flat_hbm.at[pos_v], val_v, sem2)

        @pl.loop(0, NCHUNK)
        def _(c):
            off = c * CHUNK
            pltpu.async_copy(
                table_hbm.at[idx_v.at[pl.ds(off, CHUNK)]], rows_v, sem
            ).wait()
            pltpu.sync_copy(rows_v, out_hbm.at[pl.ds(base + off, CHUNK)])

        lse_dma.wait()
        val_dma.wait()
        acc_v[...] = jnp.zeros((L,), jnp.float32)

        @pl.loop(0, BPW, step=L)
        def _(i):
            sl = pl.ds(i, L)
            acc_v[...] = acc_v[...] + (lsev_v[sl] - val_v[sl])

        pltpu.sync_copy(acc_v, part_hbm.at[wid])

    return k(table_p, emb_flat, lse_flat, idx_flat, tgt_flat)


def kernel(idx, targets, embedding):
    idx_flat = idx.reshape(-1)
    tgt_flat = targets.reshape(-1)
    emb_p = jnp.pad(embedding, ((0, 0), (0, VP - V)),
                    constant_values=jnp.float32(-1e30))
    emb_flat = embedding.reshape(-1)
    lse_flat = _tc_lse_table(embedding).reshape(-1)
    out_p, parts = _sc_gather_and_nll(emb_p, emb_flat, lse_flat,
                                      idx_flat, tgt_flat)
    return out_p[:, :V], jnp.sum(parts) / jnp.float32(N)


# speedup vs baseline: 1.6402x; 1.0160x over previous
import functools

import jax
import jax.numpy as jnp
from jax import lax
from jax.experimental import pallas as pl
from jax.experimental.pallas import tpu as pltpu
from jax.experimental.pallas import tpu_sc as plsc

V = 1000
VP = 1024
N = 51200
NC, NS = 2, 16
NW = NC * NS
BPW = N // NW
CHUNK = 40
NCHUNK = BPW // CHUNK
L = 16


def _tc_lse_table(embedding):
    def body(x_ref, o_ref):
        rows = x_ref[...]
        m = jnp.max(rows, axis=1, keepdims=True)
        s = jnp.sum(jnp.exp(rows - m), axis=1, keepdims=True)
        o_ref[...] = m + jnp.log(s)

    return pl.pallas_call(
        body,
        in_specs=[pl.BlockSpec((V, V), lambda: (0, 0))],
        out_specs=pl.BlockSpec((V, 1), lambda: (0, 0)),
        out_shape=jax.ShapeDtypeStruct((V, 1), jnp.float32),
    )(embedding)


def _sc_gather_and_nll(table_p, emb_flat, lse_flat, idx_flat, tgt_flat):
    mesh = plsc.VectorSubcoreMesh(core_axis_name="c", subcore_axis_name="s")

    @functools.partial(
        pl.kernel,
        out_type=[
            jax.ShapeDtypeStruct((N, VP), jnp.float32),
            jax.ShapeDtypeStruct((NW, L), jnp.float32),
        ],
        mesh=mesh,
        scratch_types=[
            pltpu.VMEM((BPW,), jnp.int32),
            pltpu.VMEM((BPW,), jnp.int32),
            pltpu.VMEM((BPW,), jnp.float32),
            pltpu.VMEM((BPW,), jnp.float32),
            pltpu.VMEM((CHUNK, VP), jnp.float32),
            pltpu.VMEM((CHUNK, VP), jnp.float32),
            pltpu.VMEM((L,), jnp.float32),
            pltpu.SemaphoreType.DMA,
            pltpu.SemaphoreType.DMA,
            pltpu.SemaphoreType.DMA,
        ],
    )
    def k(table_hbm, flat_hbm, lse_hbm, idx_hbm, tgt_hbm, out_hbm, part_hbm,
          idx_v, pos_v, lsev_v, val_v, buf_a, buf_b, acc_v, sem_a, sem_b,
          sem2):
        wid = lax.axis_index("s") * NC + lax.axis_index("c")
        base = wid * BPW
        pltpu.sync_copy(idx_hbm.at[pl.ds(base, BPW)], idx_v)
        pltpu.sync_copy(tgt_hbm.at[pl.ds(base, BPW)], pos_v)

        @pl.loop(0, BPW, step=L)
        def _(i):
            sl = pl.ds(i, L)
            pos_v[sl] = idx_v[sl] * V + pos_v[sl]

        lse_dma = pltpu.async_copy(lse_hbm.at[idx_v], lsev_v, sem2)
        val_dma = pltpu.async_copy(flat_hbm.at[pos_v], val_v, sem2)

        def gather(c, buf, sem):
            return pltpu.async_copy(
                table_hbm.at[idx_v.at[pl.ds(c * CHUNK, CHUNK)]], buf, sem)

        def drain(c, buf, sem):
            pltpu.make_async_copy(
                table_hbm.at[idx_v.at[pl.ds(c * CHUNK, CHUNK)]], buf, sem
            ).wait()

        def put(c, buf):
            pltpu.sync_copy(buf, out_hbm.at[pl.ds(base + c * CHUNK, CHUNK)])

        gather(0, buf_a, sem_a)

        @pl.loop(0, NCHUNK, step=2)
        def _(c):
            drain(c, buf_a, sem_a)
            gather(c + 1, buf_b, sem_b)
            put(c, buf_a)
            drain(c + 1, buf_b, sem_b)

            @pl.when(c + 2 < NCHUNK)
            def _():
                gather(c + 2, buf_a, sem_a)

            put(c + 1, buf_b)

        lse_dma.wait()
        val_dma.wait()
        acc_v[...] = jnp.zeros((L,), jnp.float32)

        @pl.loop(0, BPW, step=L)
        def _(i):
            sl = pl.ds(i, L)
            acc_v[...] = acc_v[...] + (lsev_v[sl] - val_v[sl])

        pltpu.sync_copy(acc_v, part_hbm.at[wid])

    return k(table_p, emb_flat, lse_flat, idx_flat, tgt_flat)


def kernel(idx, targets, embedding):
    idx_flat = idx.reshape(-1)
    tgt_flat = targets.reshape(-1)
    emb_p = jnp.pad(embedding, ((0, 0), (0, VP - V)),
                    constant_values=jnp.float32(-1e30))
    emb_flat = embedding.reshape(-1)
    lse_flat = _tc_lse_table(embedding).reshape(-1)
    out_p, parts = _sc_gather_and_nll(emb_p, emb_flat, lse_flat,
                                      idx_flat, tgt_flat)
    return out_p[:, :V], jnp.sum(parts) / jnp.float32(N)


# speedup vs baseline: 1.7684x; 1.0782x over previous
import functools

import jax
import jax.numpy as jnp
from jax import lax
from jax.experimental import pallas as pl
from jax.experimental.pallas import tpu as pltpu
from jax.experimental.pallas import tpu_sc as plsc

V = 1000
VP = 1024
N = 51200
NC, NS = 2, 16
NW = NC * NS
BPW = N // NW
CHUNK = 80
NCHUNK = BPW // CHUNK
L = 16


def _tc_lse_table(embedding):
    def body(x_ref, o_ref):
        rows = x_ref[...]
        m = jnp.max(rows, axis=1, keepdims=True)
        s = jnp.sum(jnp.exp(rows - m), axis=1, keepdims=True)
        o_ref[...] = m + jnp.log(s)

    return pl.pallas_call(
        body,
        in_specs=[pl.BlockSpec((V, V), lambda: (0, 0))],
        out_specs=pl.BlockSpec((V, 1), lambda: (0, 0)),
        out_shape=jax.ShapeDtypeStruct((V, 1), jnp.float32),
    )(embedding)


def _sc_gather_and_nll(table_p, lse_flat, idx_flat, tgt_flat):
    mesh = plsc.VectorSubcoreMesh(core_axis_name="c", subcore_axis_name="s")

    @functools.partial(
        pl.kernel,
        out_type=[
            jax.ShapeDtypeStruct((N, VP), jnp.float32),
            jax.ShapeDtypeStruct((NW, L), jnp.float32),
        ],
        mesh=mesh,
        compiler_params=pltpu.CompilerParams(needs_layout_passes=False),
        scratch_types=[
            pltpu.VMEM((BPW,), jnp.int32),
            pltpu.VMEM((BPW,), jnp.int32),
            pltpu.VMEM((V,), jnp.float32),
            pltpu.VMEM((CHUNK, VP), jnp.float32),
            pltpu.VMEM((L,), jnp.float32),
            pltpu.SemaphoreType.DMA,
            pltpu.SemaphoreType.DMA,
        ],
    )
    def k(table_hbm, lse_hbm, idx_hbm, tgt_hbm, out_hbm, part_hbm,
          idx_v, tgt_v, lse_v, buf, acc_v, sem_g, sem_w):
        wid = lax.axis_index("s") * NC + lax.axis_index("c")
        base = wid * BPW
        pltpu.sync_copy(idx_hbm.at[pl.ds(base, BPW)], idx_v)
        pltpu.sync_copy(tgt_hbm.at[pl.ds(base, BPW)], tgt_v)
        pltpu.sync_copy(lse_hbm, lse_v)
        acc_v[...] = jnp.zeros((L,), jnp.float32)
        row_iota = lax.iota(jnp.int32, L)

        @pl.loop(0, NCHUNK)
        def _(c):
            off = c * CHUNK

            @pl.when(c > 0)
            def _():
                pltpu.make_async_copy(
                    buf, out_hbm.at[pl.ds(base + off - CHUNK, CHUNK)], sem_w
                ).wait()

            pltpu.async_copy(
                table_hbm.at[idx_v.at[pl.ds(off, CHUNK)]], buf, sem_g
            ).wait()
            pltpu.async_copy(
                buf, out_hbm.at[pl.ds(base + off, CHUNK)], sem_w)

            @pl.loop(0, CHUNK, step=L)
            def _(j):
                t_reg = tgt_v[pl.ds(off + j, L)]
                i_reg = idx_v[pl.ds(off + j, L)]
                vals = plsc.load_gather(buf, [row_iota + j, t_reg])
                lses = plsc.load_gather(lse_v, [i_reg])
                acc_v[...] = acc_v[...] + (lses - vals)

        pltpu.make_async_copy(
            buf, out_hbm.at[pl.ds(base + BPW - CHUNK, CHUNK)], sem_w
        ).wait()
        pltpu.sync_copy(acc_v, part_hbm.at[wid])

    return k(table_p, lse_flat, idx_flat, tgt_flat)


def kernel(idx, targets, embedding):
    idx_flat = idx.reshape(-1)
    tgt_flat = targets.reshape(-1)
    emb_p = jnp.pad(embedding, ((0, 0), (0, VP - V)),
                    constant_values=jnp.float32(-1e30))
    lse_flat = _tc_lse_table(embedding).reshape(-1)
    out_p, parts = _sc_gather_and_nll(emb_p, lse_flat, idx_flat, tgt_flat)
    return out_p[:, :V], jnp.sum(parts) / jnp.float32(N)
